# Initial kernel scaffold; baseline (speedup 1.0000x reference)
import jax
import jax.numpy as jnp
from jax.experimental import pallas as pl


def kernel(boxes, scores):
    raise NotImplementedError("write your pallas kernel here")


# speedup vs baseline: 13.4266x; 13.4266x over previous
import jax
import jax.numpy as jnp
from jax import lax
from jax.experimental import pallas as pl

N_REAL = 20000
N_PAD = 20480
ROWS = 160
K = 1000
KP = 1024
OUT_K = 100
IOU_T = 0.5
NEG = -1e9
F32 = jnp.float32


def _nms_body(data_ref, s2d_ref, out_ref):
    s2d = s2d_ref[...]
    keys = lax.bitcast_convert_type(s2d, jnp.int32)

    def bs_body(_, carry):
        lo, hi = carry
        mid = lo + (hi - lo) // 2
        cnt = jnp.sum((keys > mid).astype(F32))
        big = cnt >= F32(K)
        return jnp.where(big, mid + 1, lo), jnp.where(big, hi, mid)

    T, _ = lax.fori_loop(0, 31, bs_body,
                         (jnp.int32(0), jnp.int32(0x7F800000)))
    count_above = jnp.sum((keys > T).astype(F32))
    need_eq = F32(K) - count_above

    r128 = lax.broadcasted_iota(jnp.int32, (128, 128), 0)
    c128 = lax.broadcasted_iota(jnp.int32, (128, 128), 1)
    U = (r128 <= c128).astype(F32)
    rR = lax.broadcasted_iota(jnp.int32, (ROWS, ROWS), 0)
    cR = lax.broadcasted_iota(jnp.int32, (ROWS, ROWS), 1)
    SLR = (cR < rR).astype(F32)

    def ex_prefix(m):
        incl = jnp.dot(m, U, preferred_element_type=F32)
        row_off = jnp.dot(SLR, incl[:, 127:128], preferred_element_type=F32)
        return incl - m + row_off

    eq = keys == T
    eq_ex = ex_prefix(eq.astype(F32))
    sel = (keys > T) | (eq & (eq_ex < need_eq))
    pos = ex_prefix(sel.astype(F32))
    posm = jnp.where(sel, pos, -1.0)

    lane3 = lax.broadcasted_iota(jnp.int32, (8, 128, KP), 2).astype(F32)
    cand = jnp.zeros((KP, 8), F32)
    for c in range(20):
        pb = posm[c * 8:(c + 1) * 8, :]
        oh = (pb[:, :, None] == lane3).astype(F32).reshape(KP, KP)
        dchunk = data_ref[c * KP:(c + 1) * KP, :]
        cand = cand + lax.dot_general(
            oh, dchunk, (((0,), (0,)), ((), ())), preferred_element_type=F32,
            precision=lax.Precision.HIGHEST)

    sub1 = lax.broadcasted_iota(jnp.int32, (KP, 1), 0).astype(F32)
    is_pad = sub1 >= F32(K)
    score_c = jnp.where(is_pad, -1.0, cand[:, 4:5])
    idx_c = jnp.where(is_pad, 1.0e7 + sub1, cand[:, 5:6])
    subK = lax.broadcasted_iota(jnp.int32, (KP, KP), 0)
    laneK = lax.broadcasted_iota(jnp.int32, (KP, KP), 1)
    I = (subK == laneK).astype(F32)
    score_r = lax.dot_general(score_c, I, (((0,), (0,)), ((), ())),
                              preferred_element_type=F32,
                              precision=lax.Precision.HIGHEST)
    idx_r = lax.dot_general(idx_c, I, (((0,), (0,)), ((), ())),
                            preferred_element_type=F32,
                            precision=lax.Precision.HIGHEST)
    beats = ((score_c > score_r)
             | ((score_c == score_r) & (idx_c < idx_r))).astype(F32)
    rank_r = jnp.sum(beats, axis=0, keepdims=True)
    perm = (sub1 == rank_r).astype(F32)
    cand2 = jnp.concatenate([cand[:, 0:4], score_c, idx_c, cand[:, 6:8]],
                            axis=1)
    srt = jnp.dot(perm, cand2, preferred_element_type=F32,
                  precision=lax.Precision.HIGHEST)

    bt = lax.dot_general(srt, I, (((0,), (0,)), ((), ())),
                         preferred_element_type=F32,
                         precision=lax.Precision.HIGHEST)
    x1c, y1c, x2c, y2c = (srt[:, 0:1], srt[:, 1:2], srt[:, 2:3], srt[:, 3:4])
    x1r, y1r, x2r, y2r = (bt[0:1, :], bt[1:2, :], bt[2:3, :], bt[3:4, :])
    iw = jnp.maximum(jnp.minimum(x2c, x2r) - jnp.maximum(x1c, x1r), 0.0)
    ih = jnp.maximum(jnp.minimum(y2c, y2r) - jnp.maximum(y1c, y1r), 0.0)
    inter = iw * ih
    area_c = (x2c - x1c) * (y2c - y1c)
    area_r = (x2r - x1r) * (y2r - y1r)
    union = area_c + area_r - inter
    iou = inter / jnp.maximum(union, 1e-9)
    slK = (subK < laneK).astype(F32)
    M = (iou > IOU_T).astype(F32) * slK

    def cond(carry):
        _, changed, it = carry
        return changed & (it < KP + 1)

    def step(carry):
        alive, _, it = carry
        cnt = jnp.dot(alive, M, preferred_element_type=F32)
        alive_new = (cnt <= 0.0).astype(F32)
        return alive_new, jnp.any(alive_new != alive), it + 1

    alive, _, _ = lax.while_loop(
        cond, step, (jnp.ones((1, KP), F32), jnp.bool_(True), jnp.int32(0)))

    lane1 = lax.broadcasted_iota(jnp.int32, (1, KP), 1)
    valid = (lane1 < K).astype(F32)
    alive_v = alive * valid
    dead_v = (1.0 - alive) * valid
    alive_ex = jnp.dot(alive_v, slK, preferred_element_type=F32)
    dead_ex = jnp.dot(dead_v, slK, preferred_element_type=F32)
    n_alive = jnp.sum(alive_v)
    score_sr = bt[4:5, :]
    fin_rank = jnp.where(alive_v > 0, alive_ex, n_alive + dead_ex)
    fin_rank = jnp.where(valid > 0, fin_rank, 1.0e6)
    kept_r = jnp.where(alive > 0, score_sr, NEG)
    kept_c = lax.dot_general(I, kept_r, (((1,), (1,)), ((), ())),
                             preferred_element_type=F32,
                             precision=lax.Precision.HIGHEST)
    outd = jnp.concatenate(
        [srt[:, 0:4], kept_c, jnp.zeros((KP, 3), F32)], axis=1)
    out_sub = lax.broadcasted_iota(jnp.int32, (128, 1), 0).astype(F32)
    O = (out_sub == fin_rank).astype(F32)
    out_ref[...] = jnp.dot(O, outd, preferred_element_type=F32,
                           precision=lax.Precision.HIGHEST)


def kernel(boxes, scores):
    idx = jnp.arange(N_REAL, dtype=F32)
    data = jnp.concatenate(
        [boxes, scores[:, None], idx[:, None], jnp.zeros((N_REAL, 2), F32)],
        axis=1)
    pad = jnp.concatenate(
        [jnp.zeros((N_PAD - N_REAL, 4), F32),
         jnp.full((N_PAD - N_REAL, 1), -1.0, F32),
         jnp.zeros((N_PAD - N_REAL, 3), F32)], axis=1)
    data = jnp.concatenate([data, pad], axis=0)
    s2d = data[:, 4].reshape(ROWS, 128)
    out = pl.pallas_call(
        _nms_body,
        out_shape=jax.ShapeDtypeStruct((128, 8), F32),
    )(data, s2d)
    return out[:OUT_K, :5]


# speedup vs baseline: 13.7011x; 1.0204x over previous
import jax
import jax.numpy as jnp
from jax import lax
from jax.experimental import pallas as pl
from jax.experimental.pallas import tpu as pltpu

N_REAL = 20000
N_PAD = 20480
ROWS = 160
K = 1000
KP = 1024
OUT_K = 100
IOU_T = 0.5
NEG = -1e9
F32 = jnp.float32


def _nms_body(data_ref, s2d_ref, out_ref, srt_ref, bt_ref, kill_ref,
              alive_ref):
    s2d = s2d_ref[...]
    keys = lax.bitcast_convert_type(s2d, jnp.int32)

    def bs_body(_, carry):
        lo, hi = carry
        mid = lo + (hi - lo) // 2
        cnt = jnp.sum((keys > mid).astype(F32))
        big = cnt >= F32(K)
        return jnp.where(big, mid + 1, lo), jnp.where(big, hi, mid)

    T, _ = lax.fori_loop(0, 31, bs_body,
                         (jnp.int32(0), jnp.int32(0x7F800000)))
    count_above = jnp.sum((keys > T).astype(F32))
    need_eq = F32(K) - count_above

    r128 = lax.broadcasted_iota(jnp.int32, (128, 128), 0)
    c128 = lax.broadcasted_iota(jnp.int32, (128, 128), 1)
    U = (r128 <= c128).astype(F32)
    rR = lax.broadcasted_iota(jnp.int32, (ROWS, ROWS), 0)
    cR = lax.broadcasted_iota(jnp.int32, (ROWS, ROWS), 1)
    SLR = (cR < rR).astype(F32)

    def ex_prefix(m):
        incl = jnp.dot(m, U, preferred_element_type=F32)
        row_off = jnp.dot(SLR, incl[:, 127:128], preferred_element_type=F32)
        return incl - m + row_off

    eq = keys == T
    eq_ex = ex_prefix(eq.astype(F32))
    sel = (keys > T) | (eq & (eq_ex < need_eq))
    pos = ex_prefix(sel.astype(F32))
    posm = jnp.where(sel, pos, -1.0)

    lane3 = lax.broadcasted_iota(jnp.int32, (8, 128, KP), 2).astype(F32)
    cand = jnp.zeros((KP, 8), F32)
    for c in range(20):
        pb = posm[c * 8:(c + 1) * 8, :]
        oh = (pb[:, :, None] == lane3).astype(F32).reshape(KP, KP)
        dchunk = data_ref[c * KP:(c + 1) * KP, :]
        cand = cand + lax.dot_general(
            oh, dchunk, (((0,), (0,)), ((), ())), preferred_element_type=F32,
            precision=lax.Precision.HIGHEST)

    sub1 = lax.broadcasted_iota(jnp.int32, (KP, 1), 0).astype(F32)
    is_pad = sub1 >= F32(K)
    score_c = jnp.where(is_pad, -1.0, cand[:, 4:5])
    idx_c = jnp.where(is_pad, 1.0e7 + sub1, cand[:, 5:6])
    subK = lax.broadcasted_iota(jnp.int32, (KP, KP), 0)
    laneK = lax.broadcasted_iota(jnp.int32, (KP, KP), 1)
    I = (subK == laneK).astype(F32)
    score_r = lax.dot_general(score_c, I, (((0,), (0,)), ((), ())),
                              preferred_element_type=F32,
                              precision=lax.Precision.HIGHEST)
    idx_r = lax.dot_general(idx_c, I, (((0,), (0,)), ((), ())),
                            preferred_element_type=F32,
                            precision=lax.Precision.HIGHEST)
    beats = ((score_c > score_r)
             | ((score_c == score_r) & (idx_c < idx_r))).astype(F32)
    rank_r = jnp.sum(beats, axis=0, keepdims=True)
    perm = (sub1 == rank_r).astype(F32)
    cand2 = jnp.concatenate([cand[:, 0:4], score_c, idx_c, cand[:, 6:8]],
                            axis=1)
    srt = jnp.dot(perm, cand2, preferred_element_type=F32,
                  precision=lax.Precision.HIGHEST)

    bt = lax.dot_general(srt, I, (((0,), (0,)), ((), ())),
                         preferred_element_type=F32,
                         precision=lax.Precision.HIGHEST)
    srt_ref[...] = srt
    bt_ref[...] = bt
    kill_ref[...] = jnp.zeros((1, KP), F32)
    alive_ref[...] = jnp.ones((1, KP), F32)
    x1r, y1r, x2r, y2r = (bt[0:1, :], bt[1:2, :], bt[2:3, :], bt[3:4, :])
    area_r = (x2r - x1r) * (y2r - y1r)
    lane1f = lax.broadcasted_iota(jnp.int32, (1, KP), 1).astype(F32)
    validf = (lane1f < F32(K)).astype(F32)
    BS = 128
    sub_b = lax.broadcasted_iota(jnp.int32, (BS, 1), 0)
    lane_b = lax.broadcasted_iota(jnp.int32, (1, BS), 1)
    tri_bb = (sub_b < lane_b).astype(F32)

    def blk_cond(carry):
        b, n_alive = carry
        return (b < KP // BS) & (n_alive < F32(OUT_K))

    def blk_step(carry):
        b, _ = carry
        r0 = b * BS
        blk = srt_ref[pl.ds(r0, BS), :]
        x1b, y1b, x2b, y2b = (blk[:, 0:1], blk[:, 1:2],
                              blk[:, 2:3], blk[:, 3:4])
        area_b = (x2b - x1b) * (y2b - y1b)
        iw = jnp.maximum(jnp.minimum(x2b, x2r) - jnp.maximum(x1b, x1r), 0.0)
        ih = jnp.maximum(jnp.minimum(y2b, y2r) - jnp.maximum(y1b, y1r), 0.0)
        inter = iw * ih
        union = area_b + area_r - inter
        iou = inter / jnp.maximum(union, 1e-9)
        a_glob = (sub_b + r0).astype(F32)
        E = (iou > IOU_T).astype(F32) * (a_glob < lane1f).astype(F32)
        x1w = bt_ref[0:1, pl.ds(r0, BS)]
        y1w = bt_ref[1:2, pl.ds(r0, BS)]
        x2w = bt_ref[2:3, pl.ds(r0, BS)]
        y2w = bt_ref[3:4, pl.ds(r0, BS)]
        iwb = jnp.maximum(jnp.minimum(x2b, x2w) - jnp.maximum(x1b, x1w), 0.0)
        ihb = jnp.maximum(jnp.minimum(y2b, y2w) - jnp.maximum(y1b, y1w), 0.0)
        interb = iwb * ihb
        area_w = (x2w - x1w) * (y2w - y1w)
        unionb = area_b + area_w - interb
        ioub = interb / jnp.maximum(unionb, 1e-9)
        Ebb = (ioub > IOU_T).astype(F32) * tri_bb
        base_kill = kill_ref[0:1, pl.ds(r0, BS)]

        def fp_cond(c):
            _, changed, it = c
            return changed & (it < BS + 1)

        def fp_step(c):
            ab, _, it = c
            cnt = base_kill + jnp.dot(ab, Ebb, preferred_element_type=F32)
            ab_new = (cnt <= 0.0).astype(F32)
            return ab_new, jnp.any(ab_new != ab), it + 1

        alive_b, _, _ = lax.while_loop(
            fp_cond, fp_step,
            ((base_kill <= 0.0).astype(F32), jnp.bool_(True), jnp.int32(0)))
        kill_ref[...] = kill_ref[...] + jnp.dot(alive_b, E,
                                                preferred_element_type=F32)
        alive_ref[0:1, pl.ds(r0, BS)] = alive_b
        done_mask = (lane1f < (b + 1).astype(F32) * F32(BS)).astype(F32)
        n_alive = jnp.sum(alive_ref[...] * done_mask * validf)
        return b + 1, n_alive

    lax.while_loop(blk_cond, blk_step, (jnp.int32(0), F32(0.0)))
    alive = alive_ref[...]

    slK = (subK < laneK).astype(F32)
    lane1 = lax.broadcasted_iota(jnp.int32, (1, KP), 1)
    valid = (lane1 < K).astype(F32)
    alive_v = alive * valid
    dead_v = (1.0 - alive) * valid
    alive_ex = jnp.dot(alive_v, slK, preferred_element_type=F32)
    dead_ex = jnp.dot(dead_v, slK, preferred_element_type=F32)
    n_alive = jnp.sum(alive_v)
    score_sr = bt[4:5, :]
    fin_rank = jnp.where(alive_v > 0, alive_ex, n_alive + dead_ex)
    fin_rank = jnp.where(valid > 0, fin_rank, 1.0e6)
    kept_r = jnp.where(alive > 0, score_sr, NEG)
    kept_c = lax.dot_general(I, kept_r, (((1,), (1,)), ((), ())),
                             preferred_element_type=F32,
                             precision=lax.Precision.HIGHEST)
    outd = jnp.concatenate(
        [srt[:, 0:4], kept_c, jnp.zeros((KP, 3), F32)], axis=1)
    out_sub = lax.broadcasted_iota(jnp.int32, (128, 1), 0).astype(F32)
    O = (out_sub == fin_rank).astype(F32)
    out_ref[...] = jnp.dot(O, outd, preferred_element_type=F32,
                           precision=lax.Precision.HIGHEST)


def kernel(boxes, scores):
    idx = jnp.arange(N_REAL, dtype=F32)
    data = jnp.concatenate(
        [boxes, scores[:, None], idx[:, None], jnp.zeros((N_REAL, 2), F32)],
        axis=1)
    pad = jnp.concatenate(
        [jnp.zeros((N_PAD - N_REAL, 4), F32),
         jnp.full((N_PAD - N_REAL, 1), -1.0, F32),
         jnp.zeros((N_PAD - N_REAL, 3), F32)], axis=1)
    data = jnp.concatenate([data, pad], axis=0)
    s2d = data[:, 4].reshape(ROWS, 128)
    out = pl.pallas_call(
        _nms_body,
        out_shape=jax.ShapeDtypeStruct((128, 8), F32),
        scratch_shapes=[
            pltpu.VMEM((KP, 8), F32),
            pltpu.VMEM((8, KP), F32),
            pltpu.VMEM((1, KP), F32),
            pltpu.VMEM((1, KP), F32),
        ],
    )(data, s2d)
    return out[:OUT_K, :5]
